# Initial kernel scaffold; baseline (speedup 1.0000x reference)
import jax
import jax.numpy as jnp
from jax.experimental import pallas as pl


def kernel(x, edge_index, W1, b1, theta1, W2, b2, theta2):
    raise NotImplementedError("write your pallas kernel here")


# speedup vs baseline: 8.0937x; 8.0937x over previous
import functools

import jax
import jax.numpy as jnp
from jax.experimental import pallas as pl

N = 10000
E = 160000
F_IN = 128
HEADS = 8
HIDDEN = 64
CLASSES = 40
ORDER = 16


def _mm_kernel(x_ref, w_ref, o_ref):
    o_ref[...] = jnp.dot(x_ref[...], w_ref[...],
                         preferred_element_type=jnp.float32)


def _mm(x, w, bm=1000):
    m, k = x.shape
    k2, n = w.shape
    return pl.pallas_call(
        _mm_kernel,
        out_shape=jax.ShapeDtypeStruct((m, n), jnp.float32),
        grid=(m // bm,),
        in_specs=[
            pl.BlockSpec((bm, k), lambda i: (i, 0)),
            pl.BlockSpec((k2, n), lambda i: (0, 0)),
        ],
        out_specs=pl.BlockSpec((bm, n), lambda i: (i, 0)),
    )(x, w)


def kernel(x, edge_index, W1, b1, theta1, W2, b2, theta2):
    src = edge_index[0]
    dst = edge_index[1]
    ones = jnp.ones((E,), dtype=jnp.float32)
    deg = (jax.ops.segment_sum(ones, dst, num_segments=N)
           + jax.ops.segment_sum(ones, src, num_segments=N))
    deg = jnp.maximum(deg, 1.0)
    d = jax.lax.rsqrt(deg)
    dsq = d * d
    sd = jnp.sqrt(deg)

    def cheb_G(h, theta_feat):
        G_prev = d[:, None] * h
        S = jax.ops.segment_sum(G_prev[src], dst, num_segments=N)
        G_cur = -dsq[:, None] * S
        accG = (theta_feat[None, :, 0] * G_prev
                + theta_feat[None, :, 1] * G_cur)
        for k in range(2, ORDER + 1):
            S = jax.ops.segment_sum(G_cur[src], dst, num_segments=N)
            G_next = -2.0 * dsq[:, None] * S - G_prev
            accG = accG + theta_feat[None, :, k] * G_next
            G_prev, G_cur = G_cur, G_next
        return sd[:, None] * accG

    th1 = jnp.repeat(theta1, HIDDEN, axis=0)
    th2 = jnp.broadcast_to(theta2, (CLASSES, ORDER + 1))

    h1 = _mm(x, W1)
    f1 = cheb_G(h1, th1)
    layer1 = jax.nn.elu(f1 + b1)
    h2 = _mm(layer1, W2.reshape(HEADS * HIDDEN, CLASSES))
    f2 = cheb_G(h2, th2) + b2
    layer2 = f2
    out = jax.nn.elu(layer2)
    logp = jax.nn.log_softmax(out, axis=1)
    return (logp, layer2, layer1)


# speedup vs baseline: 23.7708x; 2.9369x over previous
import functools

import jax
import jax.numpy as jnp
from jax import lax
from jax.experimental import pallas as pl
from jax.experimental.pallas import tpu as pltpu
from jax.experimental.pallas import tpu_sc as plsc

N = 10000
E = 160000
F_IN = 128
HEADS = 8
HIDDEN = 64
CLASSES = 40
ORDER = 16

NSC = 2
NSUB = 16
NPAD = 10240
RPS = NPAD // NSUB
ST = RPS // 128
B = 128
EPS = E // NSUB
NB = -(-EPS // B)
JUNK = N

_f32 = jnp.float32
_mesh = plsc.VectorSubcoreMesh(core_axis_name="c", subcore_axis_name="s")


def _fill_zero(zbuf, nv):
    zero = jnp.zeros((16,), _f32)

    @pl.loop(0, 128)
    def _(i):
        for v in range(nv):
            zbuf[i, pl.ds(16 * v, 16)] = zero


def _cheb_body(nch, chunk, kind, *refs):
    nv = chunk // 16
    rounds = nch // NSC
    if kind == "first":
        (h_hbm, d_hbm, dsq_hbm, tha_hbm, thb_hbm, src_hbm, dst_hbm,
         g0_hbm, gn_hbm, acc_hbm,
         src_v, dst_v, rows_v, zbuf, sv, gv, av, dsq_v, aux_v, th_a, th_b,
         s_sp) = refs
    elif kind == "mid":
        (gcur_hbm, gprev_hbm, accin_hbm, dsq_hbm, tha_hbm, src_hbm, dst_hbm,
         gn_hbm, acc_hbm,
         src_v, dst_v, rows_v, zbuf, sv, gv, av, dsq_v, aux_v, th_a, th_b,
         s_sp) = refs
    else:
        (gcur_hbm, gprev_hbm, accin_hbm, dsq_hbm, sd_hbm, tha_hbm, src_hbm,
         dst_hbm,
         acc_hbm,
         src_v, dst_v, rows_v, zbuf, sv, gv, av, dsq_v, aux_v, th_a, th_b,
         s_sp) = refs

    c = lax.axis_index("c")
    s = lax.axis_index("s")
    rbase = s * RPS

    pltpu.sync_copy(dsq_hbm.at[pl.ds(rbase, RPS)], dsq_v)
    if kind == "first":
        pltpu.sync_copy(d_hbm.at[pl.ds(rbase, RPS)], aux_v)
    elif kind == "final":
        pltpu.sync_copy(sd_hbm.at[pl.ds(rbase, RPS)], aux_v)
    lane16 = pl.ds(0, 16)
    pltpu.sync_copy(dst_hbm.at[s], dst_v)
    pltpu.sync_copy(tha_hbm, th_a)
    if kind == "first":
        pltpu.sync_copy(thb_hbm, th_b)
    _fill_zero(zbuf, nv)

    coef = -1.0 if kind == "first" else -2.0
    for r in range(rounds):
        ch = c * rounds + r
        gb = ch * NPAD
        pltpu.sync_copy(src_hbm.at[ch, s], src_v)
        tva = [th_a[ch, pl.ds(16 * v, 16)] for v in range(nv)]
        if kind == "first":
            tvb = [th_b[ch, pl.ds(16 * v, 16)] for v in range(nv)]

        if kind == "first":
            for t in range(ST):
                r0 = rbase + t * 128
                pltpu.sync_copy(h_hbm.at[pl.ds(gb + r0, 128)], gv)

                @pl.loop(0, 128)
                def _(i, t=t):
                    dd = aux_v[t * 128 + i, lane16]
                    for v in range(nv):
                        cs = pl.ds(16 * v, 16)
                        gv[i, cs] = dd * gv[i, cs]

                pltpu.sync_copy(gv, g0_hbm.at[pl.ds(gb + r0, 128)])

        for t in range(ST):
            pltpu.sync_copy(zbuf, s_sp.at[pl.ds(rbase + t * 128, 128)])
        plsc.subcore_barrier()

        gsrc = g0_hbm if kind == "first" else gcur_hbm

        @pl.loop(0, NB)
        def _(j):
            pltpu.sync_copy(gsrc.at[src_v.at[j]], rows_v)
            pltpu.sync_copy(rows_v, s_sp.at[dst_v.at[j]], add=True)

        plsc.subcore_barrier()

        for t in range(ST):
            r0 = rbase + t * 128
            g0r = gb + r0
            pltpu.sync_copy(s_sp.at[pl.ds(r0, 128)], sv)
            gp_src = g0_hbm if kind == "first" else gprev_hbm
            pltpu.sync_copy(gp_src.at[pl.ds(g0r, 128)], gv)
            if kind != "first":
                pltpu.sync_copy(accin_hbm.at[pl.ds(g0r, 128)], av)

            @pl.loop(0, 128)
            def _(i, t=t):
                m = coef * dsq_v[t * 128 + i, lane16]
                if kind == "final":
                    sdd = aux_v[t * 128 + i, lane16]
                for v in range(nv):
                    cs = pl.ds(16 * v, 16)
                    gn = m * sv[i, cs]
                    if kind == "first":
                        a = tva[v] * gv[i, cs] + tvb[v] * gn
                    else:
                        gn = gn - gv[i, cs]
                        a = av[i, cs] + tva[v] * gn
                    if kind == "final":
                        a = sdd * a
                    av[i, cs] = a
                    if kind != "final":
                        gv[i, cs] = gn

            if kind != "final":
                pltpu.sync_copy(gv, gn_hbm.at[pl.ds(g0r, 128)])
            pltpu.sync_copy(av, acc_hbm.at[pl.ds(g0r, 128)])
        plsc.subcore_barrier()


def _make_cheb_step(nch, chunk, kind):
    R = nch * NPAD
    n_out = {"first": 3, "mid": 2, "final": 1}[kind]
    outs = tuple(jax.ShapeDtypeStruct((R, chunk), _f32) for _ in range(n_out))
    scratch = (
        pltpu.VMEM((NB, B), jnp.int32),
        pltpu.VMEM((NB, B), jnp.int32),
        pltpu.VMEM((B, chunk), _f32),
        pltpu.VMEM((128, chunk), _f32),
        pltpu.VMEM((128, chunk), _f32),
        pltpu.VMEM((128, chunk), _f32),
        pltpu.VMEM((128, chunk), _f32),
        pltpu.VMEM((RPS, 16), _f32),
        pltpu.VMEM((RPS, 16), _f32),
        pltpu.VMEM((nch, chunk), _f32),
        pltpu.VMEM((nch, chunk), _f32),
        pltpu.VMEM_SHARED((NPAD, chunk), _f32),
    )
    return pl.kernel(
        functools.partial(_cheb_body, nch, chunk, kind),
        out_type=outs,
        mesh=_mesh,
        scratch_types=scratch,
        compiler_params=pltpu.CompilerParams(use_tc_tiling_on_sc=False),
    )


def _deg_body(*refs):
    (idx_hbm, deg_hbm, idx_v, ones_v, zbuf, obuf, s_sp) = refs
    c = lax.axis_index("c")
    s = lax.axis_index("s")
    rbase = s * RPS
    pltpu.sync_copy(idx_hbm.at[c, s], idx_v)

    one = jnp.ones((16,), _f32)

    @pl.loop(0, B)
    def _(i):
        ones_v[i, pl.ds(0, 16)] = one

    _fill_zero(zbuf, 1)
    for t in range(ST):
        pltpu.sync_copy(zbuf, s_sp.at[pl.ds(rbase + t * 128, 128)])
    plsc.subcore_barrier()

    @pl.loop(0, NB)
    def _(j):
        pltpu.sync_copy(ones_v, s_sp.at[idx_v.at[j]], add=True)

    plsc.subcore_barrier()
    for t in range(ST):
        r0 = rbase + t * 128
        pltpu.sync_copy(s_sp.at[pl.ds(r0, 128)], obuf)
        pltpu.sync_copy(obuf, deg_hbm.at[c, pl.ds(r0, 128)])


_deg_kernel = pl.kernel(
    _deg_body,
    out_type=jax.ShapeDtypeStruct((NSC, NPAD, 16), _f32),
    mesh=_mesh,
    scratch_types=(
        pltpu.VMEM((NB, B), jnp.int32),
        pltpu.VMEM((B, 16), _f32),
        pltpu.VMEM((128, 16), _f32),
        pltpu.VMEM((128, 16), _f32),
        pltpu.VMEM_SHARED((NPAD, 16), _f32),
    ),
    compiler_params=pltpu.CompilerParams(use_tc_tiling_on_sc=False),
)


def _mm1_body(x_ref, w_ref, o_ref):
    o_ref[...] = jnp.dot(x_ref[...], w_ref[...],
                         preferred_element_type=_f32)


def _mm1(x_pad, W1):
    bm = 1024
    nb = NPAD // bm
    return pl.pallas_call(
        _mm1_body,
        out_shape=jax.ShapeDtypeStruct((4 * NPAD, 128), _f32),
        grid=(nb, 4),
        in_specs=[
            pl.BlockSpec((bm, F_IN), lambda i, ch: (i, 0)),
            pl.BlockSpec((F_IN, 128), lambda i, ch: (0, ch)),
        ],
        out_specs=pl.BlockSpec((bm, 128), lambda i, ch, nb=nb: (ch * nb + i, 0)),
    )(x_pad, W1)


def _mid_body(acc_ref, b_ref, w_ref, l1_ref, h2_ref):
    a = acc_ref[...] + b_ref[0:1, :]
    l1 = jnp.where(a > 0, a, jnp.exp(a) - 1.0)
    l1_ref[...] = l1

    @pl.when(pl.program_id(1) == 0)
    def _():
        h2_ref[...] = jnp.zeros_like(h2_ref)

    h2_ref[...] += jnp.dot(l1, w_ref[...], preferred_element_type=_f32)


def _mid(accT1, b1_2d, W2pad):
    bm = 1024
    nb = NPAD // bm
    return pl.pallas_call(
        _mid_body,
        out_shape=(
            jax.ShapeDtypeStruct((NPAD, HEADS * HIDDEN), _f32),
            jax.ShapeDtypeStruct((NPAD, 128), _f32),
        ),
        grid=(nb, 4),
        in_specs=[
            pl.BlockSpec((bm, 128), lambda i, ch, nb=nb: (ch * nb + i, 0)),
            pl.BlockSpec((8, 128), lambda i, ch: (ch, 0)),
            pl.BlockSpec((128, 128), lambda i, ch: (ch, 0)),
        ],
        out_specs=(
            pl.BlockSpec((bm, 128), lambda i, ch: (i, ch)),
            pl.BlockSpec((bm, 128), lambda i, ch: (i, 0)),
        ),
    )(accT1, b1_2d, W2pad)


def _final_body(x_ref, o_ref):
    x = x_ref[...]
    e = jnp.where(x > 0, x, jnp.exp(x) - 1.0)
    col = lax.broadcasted_iota(jnp.int32, x.shape, 1)
    valid = col < CLASSES
    em = jnp.where(valid, e, -1e30)
    m = jnp.max(em, axis=1, keepdims=True)
    z = jnp.where(valid, jnp.exp(em - m), 0.0)
    lse = jnp.log(jnp.sum(z, axis=1, keepdims=True))
    o_ref[...] = em - m - lse


def _final(l2pad):
    bm = 1024
    return pl.pallas_call(
        _final_body,
        out_shape=jax.ShapeDtypeStruct((NPAD, 128), _f32),
        grid=(NPAD // bm,),
        in_specs=[pl.BlockSpec((bm, 128), lambda i: (i, 0))],
        out_specs=pl.BlockSpec((bm, 128), lambda i: (i, 0)),
    )(l2pad)


_step1_l1 = _make_cheb_step(8, 64, "first")
_step_l1 = _make_cheb_step(8, 64, "mid")
_stepF_l1 = _make_cheb_step(8, 64, "final")
_step1_l2 = _make_cheb_step(2, 32, "first")
_step_l2 = _make_cheb_step(2, 32, "mid")
_stepF_l2 = _make_cheb_step(2, 32, "final")


def _cheb_sc(h_chunk, nch, d_pad, dsq_pad, sd_pad, th, src_off, dstp,
             first_fn, mid_fn, final_fn):
    g0, g1, acc = first_fn(h_chunk, d_pad, dsq_pad, th[0], th[1],
                           src_off, dstp)
    gprev, gcur = g0, g1
    for k in range(2, ORDER):
        gn, acc = mid_fn(gcur, gprev, acc, dsq_pad, th[k], src_off, dstp)
        gprev, gcur = gcur, gn
    (accT,) = final_fn(gcur, gprev, acc, dsq_pad, sd_pad, th[ORDER],
                       src_off, dstp)
    return accT


def kernel(x, edge_index, W1, b1, theta1, W2, b2, theta2):
    src = edge_index[0]
    dst = edge_index[1]
    padw = NB * B - EPS

    srcp = jnp.pad(src.reshape(NSUB, EPS), ((0, 0), (0, padw)),
                   constant_values=JUNK).reshape(NSUB, NB, B)
    dstp = jnp.pad(dst.reshape(NSUB, EPS), ((0, 0), (0, padw)),
                   constant_values=JUNK).reshape(NSUB, NB, B)
    src1 = srcp[None] + (jnp.arange(8, dtype=jnp.int32) * NPAD)[:, None, None, None]
    src2 = srcp[None] + (jnp.arange(2, dtype=jnp.int32) * NPAD)[:, None, None, None]
    alli = jnp.pad(jnp.concatenate([src, dst]).reshape(NSC, NSUB, EPS),
                   ((0, 0), (0, 0), (0, padw)),
                   constant_values=JUNK).reshape(NSC, NSUB, NB, B)

    deg16 = _deg_kernel(alli)
    deg = jnp.maximum(deg16[0, :N, 0] + deg16[1, :N, 0], 1.0)
    d0 = lax.rsqrt(deg)

    def _bc16(v):
        return jnp.broadcast_to(jnp.pad(v, (0, NPAD - N))[:, None], (NPAD, 16))

    d_pad = _bc16(d0)
    dsq_pad = _bc16(d0 * d0)
    sd_pad = _bc16(jnp.sqrt(deg))

    th1 = jnp.repeat(theta1, HIDDEN, axis=0).T.reshape(ORDER + 1, 8, 64)
    th2 = jnp.broadcast_to(theta2.T, (ORDER + 1, 64)).reshape(ORDER + 1, 2, 32)

    x_pad = jnp.pad(x, ((0, NPAD - N), (0, 0)))
    h1c128 = _mm1(x_pad, W1)
    h1c = (h1c128.reshape(4, NPAD, 2, 64).transpose(0, 2, 1, 3)
           .reshape(8 * NPAD, 64))
    accT1 = _cheb_sc(h1c, 8, d_pad, dsq_pad, sd_pad, th1, src1, dstp,
                     _step1_l1, _step_l1, _stepF_l1)
    accT1c = (accT1.reshape(4, 2, NPAD, 64).transpose(0, 2, 1, 3)
              .reshape(4 * NPAD, 128))

    b1_2d = jnp.broadcast_to(b1.reshape(4, 1, 128), (4, 8, 128)).reshape(32, 128)
    W2pad = jnp.pad(W2.reshape(HEADS * HIDDEN, CLASSES),
                    ((0, 0), (0, 128 - CLASSES)))
    layer1_pad, h2 = _mid(accT1c, b1_2d, W2pad)
    layer1 = layer1_pad[:N]

    h2c = h2[:, :64].reshape(NPAD, 2, 32).transpose(1, 0, 2).reshape(2 * NPAD, 32)
    accT2 = _cheb_sc(h2c, 2, d_pad, dsq_pad, sd_pad, th2, src2, dstp,
                     _step1_l2, _step_l2, _stepF_l2)
    accT2_std = accT2.reshape(2, NPAD, 32).transpose(1, 0, 2).reshape(NPAD, 64)
    layer2 = accT2_std[:N, :CLASSES] + b2

    l2pad = jnp.pad(accT2_std, ((0, 0), (0, 64)))
    l2pad = l2pad + jnp.pad(b2, (0, 88))[None, :]
    logp = _final(l2pad)[:N, :CLASSES]
    return (logp, layer2, layer1)
